# baseline (speedup 1.0000x reference)
import jax
import jax.numpy as jnp
from jax import lax
from jax.experimental import pallas as pl
from jax.experimental.pallas import tpu as pltpu
from jax.experimental.pallas import tpu_sc as plsc

B = 16384
D = 128
NC, NS, L = 2, 16, 16
NW = NC * NS
BPW = B // NW
CH = 64
NCHW = BPW // CH
DEPTH = 4


def _sc_body(user_h, item_h, tag_h, ub_h, ib_h, ue_h, ie_h, te_h, out_h,
             uidx, iidx, tidx, urows, irows, trows, ubv, ibv, scorev,
             gsem, bsem, isem, osem):
    wid = lax.axis_index("s") * NC + lax.axis_index("c")
    base = wid * BPW
    rowbase = wid * NCHW

    def gather_descs(c, slot):
        return (
            pltpu.make_async_copy(ue_h.at[uidx.at[c]], urows.at[slot], gsem.at[slot]),
            pltpu.make_async_copy(ie_h.at[iidx.at[c]], irows.at[slot], gsem.at[slot]),
            pltpu.make_async_copy(te_h.at[tidx.at[c]], trows.at[slot], gsem.at[slot]),
        )

    def bias_descs(c):
        return (
            pltpu.make_async_copy(ub_h.at[uidx.at[c]], ubv.at[c], bsem),
            pltpu.make_async_copy(ib_h.at[iidx.at[c]], ibv.at[c], bsem),
        )

    idx_stage = (
        pltpu.make_async_copy(user_h.at[pl.ds(rowbase, NCHW)], uidx, isem),
        pltpu.make_async_copy(item_h.at[pl.ds(rowbase, NCHW)], iidx, isem),
        pltpu.make_async_copy(tag_h.at[pl.ds(rowbase, NCHW)], tidx, isem),
    )
    for d in idx_stage:
        d.start()
    for d in idx_stage:
        d.wait()
    for c in range(NCHW):
        for d in bias_descs(c):
            d.start()
    for c in range(DEPTH):
        for d in gather_descs(c, c):
            d.start()

    for i in range(NCHW):
        slot = i % DEPTH
        for d in gather_descs(i, slot):
            d.wait()
        for d in bias_descs(i):
            d.wait()

        def group(g, carry):
            lane = lax.iota(jnp.int32, L)

            def row(rr, acc):
                r = g * L + rr
                dv = urows[slot, r, pl.ds(0, L)] * (
                    irows[slot, r, pl.ds(0, L)] + trows[slot, r, pl.ds(0, L)])
                for k in range(1, D // L):
                    dv = dv + urows[slot, r, pl.ds(k * L, L)] * (
                        irows[slot, r, pl.ds(k * L, L)] + trows[slot, r, pl.ds(k * L, L)])
                return jnp.where(lane == rr, jnp.sum(dv), acc)

            acc = lax.fori_loop(0, L, row, jnp.zeros((L,), jnp.float32))
            scorev[i, pl.ds(g * L, L)] = (
                acc + ubv[i, pl.ds(g * L, L)] + ibv[i, pl.ds(g * L, L)]
            )
            return carry

        lax.fori_loop(0, CH // L, group, 0)
        if i + DEPTH < NCHW:
            for d in gather_descs(i + DEPTH, slot):
                d.start()
        pltpu.make_async_copy(
            scorev.at[i], out_h.at[pl.ds(base + i * CH, CH)], osem
        ).start()

    for i in range(NCHW):
        pltpu.make_async_copy(
            scorev.at[i], out_h.at[pl.ds(base + i * CH, CH)], osem
        ).wait()


def kernel(user, item, it_in, it_off, u_bias_w, i_bias_w, u_embed_w, i_embed_w, t_embed_w):
    del it_off
    ub = u_bias_w.reshape(-1)
    ib = i_bias_w.reshape(-1)
    user2 = user.reshape(B // CH, CH)
    item2 = item.reshape(B // CH, CH)
    tag2 = it_in.reshape(B // CH, CH)
    mesh = plsc.VectorSubcoreMesh(core_axis_name="c", subcore_axis_name="s")
    run = pl.kernel(
        _sc_body,
        out_type=jax.ShapeDtypeStruct((B,), jnp.float32),
        mesh=mesh,
        compiler_params=pltpu.CompilerParams(needs_layout_passes=False),
        scratch_types=[
            pltpu.VMEM((NCHW, CH), jnp.int32),
            pltpu.VMEM((NCHW, CH), jnp.int32),
            pltpu.VMEM((NCHW, CH), jnp.int32),
            pltpu.VMEM((DEPTH, CH, D), jnp.float32),
            pltpu.VMEM((DEPTH, CH, D), jnp.float32),
            pltpu.VMEM((DEPTH, CH, D), jnp.float32),
            pltpu.VMEM((NCHW, CH), jnp.float32),
            pltpu.VMEM((NCHW, CH), jnp.float32),
            pltpu.VMEM((NCHW, CH), jnp.float32),
            pltpu.SemaphoreType.DMA((DEPTH,)),
            pltpu.SemaphoreType.DMA,
            pltpu.SemaphoreType.DMA,
            pltpu.SemaphoreType.DMA,
        ],
    )
    return run(user2, item2, tag2, ub, ib, u_embed_w, i_embed_w, t_embed_w)


# speedup vs baseline: 1.0689x; 1.0689x over previous
import jax
import jax.numpy as jnp
from jax import lax
from jax.experimental import pallas as pl
from jax.experimental.pallas import tpu as pltpu
from jax.experimental.pallas import tpu_sc as plsc

B = 16384
D = 128
NC, NS, L = 2, 16, 16
NW = NC * NS
BPW = B // NW
CH = 128
NCH = BPW // CH
DEPTH = 2


def _sc_body(user_h, item_h, tag_h, ub_h, ib_h, ue_h, ie_h, te_h, out_h,
             uidx, iidx, tidx, urows, irows, trows, ubv, ibv, scorev,
             gsem, bsem, isem, osem):
    wid = lax.axis_index("s") * NC + lax.axis_index("c")
    rowbase = wid * NCH

    def gather_descs(c, slot):
        return (
            pltpu.make_async_copy(ue_h.at[uidx.at[c]], urows.at[slot], gsem.at[slot]),
            pltpu.make_async_copy(ie_h.at[iidx.at[c]], irows.at[slot], gsem.at[slot]),
            pltpu.make_async_copy(te_h.at[tidx.at[c]], trows.at[slot], gsem.at[slot]),
        )

    def bias_descs(c):
        return (
            pltpu.make_async_copy(ub_h.at[uidx.at[c]], ubv.at[c], bsem),
            pltpu.make_async_copy(ib_h.at[iidx.at[c]], ibv.at[c], bsem),
        )

    idx_stage = (
        pltpu.make_async_copy(user_h.at[pl.ds(rowbase, NCH)], uidx, isem),
        pltpu.make_async_copy(item_h.at[pl.ds(rowbase, NCH)], iidx, isem),
        pltpu.make_async_copy(tag_h.at[pl.ds(rowbase, NCH)], tidx, isem),
    )
    for d in idx_stage:
        d.start()
    for d in idx_stage:
        d.wait()
    for c in range(NCH):
        for d in bias_descs(c):
            d.start()
    for c in range(DEPTH):
        for d in gather_descs(c, c):
            d.start()

    for i in range(NCH):
        slot = i % DEPTH
        for d in gather_descs(i, slot):
            d.wait()
        for d in bias_descs(i):
            d.wait()

        def group(g, carry):
            lane = lax.iota(jnp.int32, L)

            def row(rr, acc):
                r = g * L + rr
                dv = urows[slot, r, pl.ds(0, L)] * (
                    irows[slot, r, pl.ds(0, L)] + trows[slot, r, pl.ds(0, L)])
                for k in range(1, D // L):
                    dv = dv + urows[slot, r, pl.ds(k * L, L)] * (
                        irows[slot, r, pl.ds(k * L, L)] + trows[slot, r, pl.ds(k * L, L)])
                return jnp.where(lane == rr, jnp.sum(dv), acc)

            acc = lax.fori_loop(0, L, row, jnp.zeros((L,), jnp.float32))
            scorev[i, pl.ds(g * L, L)] = (
                acc + ubv[i, pl.ds(g * L, L)] + ibv[i, pl.ds(g * L, L)]
            )
            return carry

        lax.fori_loop(0, CH // L, group, 0)
        if i + DEPTH < NCH:
            for d in gather_descs(i + DEPTH, slot):
                d.start()

    out_copy = pltpu.make_async_copy(scorev, out_h.at[pl.ds(rowbase, NCH)], osem)
    out_copy.start()
    out_copy.wait()


def kernel(user, item, it_in, it_off, u_bias_w, i_bias_w, u_embed_w, i_embed_w, t_embed_w):
    del it_off
    ub = u_bias_w.reshape(-1)
    ib = i_bias_w.reshape(-1)
    user2 = user.reshape(B // CH, CH)
    item2 = item.reshape(B // CH, CH)
    tag2 = it_in.reshape(B // CH, CH)
    mesh = plsc.VectorSubcoreMesh(core_axis_name="c", subcore_axis_name="s")
    run = pl.kernel(
        _sc_body,
        out_type=jax.ShapeDtypeStruct((B // CH, CH), jnp.float32),
        mesh=mesh,
        compiler_params=pltpu.CompilerParams(needs_layout_passes=False),
        scratch_types=[
            pltpu.VMEM((NCH, CH), jnp.int32),
            pltpu.VMEM((NCH, CH), jnp.int32),
            pltpu.VMEM((NCH, CH), jnp.int32),
            pltpu.VMEM((DEPTH, CH, D), jnp.float32),
            pltpu.VMEM((DEPTH, CH, D), jnp.float32),
            pltpu.VMEM((DEPTH, CH, D), jnp.float32),
            pltpu.VMEM((NCH, CH), jnp.float32),
            pltpu.VMEM((NCH, CH), jnp.float32),
            pltpu.VMEM((NCH, CH), jnp.float32),
            pltpu.SemaphoreType.DMA((DEPTH,)),
            pltpu.SemaphoreType.DMA,
            pltpu.SemaphoreType.DMA,
            pltpu.SemaphoreType.DMA,
        ],
    )
    out2 = run(user2, item2, tag2, ub, ib, u_embed_w, i_embed_w, t_embed_w)
    return out2.reshape(B)


# speedup vs baseline: 1.0772x; 1.0078x over previous
import jax
import jax.numpy as jnp
from jax import lax
from jax.experimental import pallas as pl
from jax.experimental.pallas import tpu as pltpu
from jax.experimental.pallas import tpu_sc as plsc

B = 16384
D = 128
NC, NS, L = 2, 16, 16
NW = NC * NS
BPW = B // NW
CH = 128
NCH = BPW // CH
DEPTH = 2


def _sc_body(user_h, item_h, tag_h, ub_h, ib_h, ue_h, ie_h, te_h, out_h,
             uidx, iidx, tidx, urows, irows, trows, ubv, ibv, scorev,
             gsem, bsem, isem, osem):
    wid = lax.axis_index("s") * NC + lax.axis_index("c")
    rowbase = wid * NCH

    def gather_descs(c, slot):
        return (
            pltpu.make_async_copy(ue_h.at[uidx.at[c]], urows.at[slot], gsem.at[slot]),
            pltpu.make_async_copy(ie_h.at[iidx.at[c]], irows.at[slot], gsem.at[slot]),
            pltpu.make_async_copy(te_h.at[tidx.at[c]], trows.at[slot], gsem.at[slot]),
        )

    def bias_descs(c):
        return (
            pltpu.make_async_copy(ub_h.at[uidx.at[c]], ubv.at[c], bsem),
            pltpu.make_async_copy(ib_h.at[iidx.at[c]], ibv.at[c], bsem),
        )

    idx_stage = (
        pltpu.make_async_copy(user_h.at[pl.ds(rowbase, NCH)], uidx, isem),
        pltpu.make_async_copy(item_h.at[pl.ds(rowbase, NCH)], iidx, isem),
        pltpu.make_async_copy(tag_h.at[pl.ds(rowbase, NCH)], tidx, isem),
    )
    for d in idx_stage:
        d.start()
    for d in idx_stage:
        d.wait()
    for c in range(DEPTH):
        for d in gather_descs(c, c):
            d.start()
    for c in range(NCH):
        for d in bias_descs(c):
            d.start()

    for i in range(NCH):
        slot = i % DEPTH
        for d in gather_descs(i, slot):
            d.wait()
        for d in bias_descs(i):
            d.wait()

        def group(g, carry):
            lane = lax.iota(jnp.int32, L)

            def row(rr, acc):
                r = g * L + rr

                def term(k):
                    return urows[slot, r, pl.ds(k * L, L)] * (
                        irows[slot, r, pl.ds(k * L, L)] + trows[slot, r, pl.ds(k * L, L)])

                dv0 = term(0)
                dv1 = term(1)
                for k in range(2, D // L, 2):
                    dv0 = dv0 + term(k)
                    dv1 = dv1 + term(k + 1)
                return jnp.where(lane == rr, jnp.sum(dv0 + dv1), acc)

            acc = lax.fori_loop(0, L, row, jnp.zeros((L,), jnp.float32))
            scorev[i, pl.ds(g * L, L)] = (
                acc + ubv[i, pl.ds(g * L, L)] + ibv[i, pl.ds(g * L, L)]
            )
            return carry

        lax.fori_loop(0, CH // L, group, 0)
        if i + DEPTH < NCH:
            for d in gather_descs(i + DEPTH, slot):
                d.start()

    out_copy = pltpu.make_async_copy(scorev, out_h.at[pl.ds(rowbase, NCH)], osem)
    out_copy.start()
    out_copy.wait()


def kernel(user, item, it_in, it_off, u_bias_w, i_bias_w, u_embed_w, i_embed_w, t_embed_w):
    del it_off
    ub = u_bias_w.reshape(-1)
    ib = i_bias_w.reshape(-1)
    user2 = user.reshape(B // CH, CH)
    item2 = item.reshape(B // CH, CH)
    tag2 = it_in.reshape(B // CH, CH)
    mesh = plsc.VectorSubcoreMesh(core_axis_name="c", subcore_axis_name="s")
    run = pl.kernel(
        _sc_body,
        out_type=jax.ShapeDtypeStruct((B // CH, CH), jnp.float32),
        mesh=mesh,
        compiler_params=pltpu.CompilerParams(needs_layout_passes=False),
        scratch_types=[
            pltpu.VMEM((NCH, CH), jnp.int32),
            pltpu.VMEM((NCH, CH), jnp.int32),
            pltpu.VMEM((NCH, CH), jnp.int32),
            pltpu.VMEM((DEPTH, CH, D), jnp.float32),
            pltpu.VMEM((DEPTH, CH, D), jnp.float32),
            pltpu.VMEM((DEPTH, CH, D), jnp.float32),
            pltpu.VMEM((NCH, CH), jnp.float32),
            pltpu.VMEM((NCH, CH), jnp.float32),
            pltpu.VMEM((NCH, CH), jnp.float32),
            pltpu.SemaphoreType.DMA((DEPTH,)),
            pltpu.SemaphoreType.DMA,
            pltpu.SemaphoreType.DMA,
            pltpu.SemaphoreType.DMA,
        ],
    )
    out2 = run(user2, item2, tag2, ub, ib, u_embed_w, i_embed_w, t_embed_w)
    return out2.reshape(B)
